# Initial kernel scaffold; baseline (speedup 1.0000x reference)
import jax
import jax.numpy as jnp
from jax.experimental import pallas as pl


def kernel(x, edge_index, W1, b1, g1, be1, W2, b2, g2, be2, W3, b3, g3, be3, fcW, fcb):
    raise NotImplementedError("write your pallas kernel here")


# speedup vs baseline: 10.2150x; 10.2150x over previous
import functools
import math

import jax
import jax.numpy as jnp
from jax import lax
from jax.experimental import pallas as pl
from jax.experimental.pallas import tpu as pltpu
from jax.experimental.pallas import tpu_sc as plsc

N = 10000
D = 128
EPS = 1e-05
NPAD = 10240
NC, NS = 2, 16
NW = NC * NS
CHUNK = 128
ROWS_PER_TILE = NPAD // NS
BLK = 1024
GRID = NPAD // BLK
BN_C = 1.0 / math.sqrt(1.0 + EPS)


def _mesh():
    return plsc.VectorSubcoreMesh(
        core_axis_name="c", subcore_axis_name="s", num_cores=NC, num_subcores=NS
    )


def _fill(buf, value):
    rows = buf.shape[0]
    vec = jnp.full((16,), value, jnp.float32)

    def row(i, carry):
        for l in range(D // 16):
            buf[i, pl.ds(l * 16, 16)] = vec
        return carry

    lax.fori_loop(0, rows, row, 0)


def _zero_acc_and_barrier(rows_v, acc, s):
    _fill(rows_v, 0.0)
    for r in range(ROWS_PER_TILE // CHUNK):
        pltpu.sync_copy(
            rows_v, acc.at[pl.ds(s * ROWS_PER_TILE + r * CHUNK, CHUNK)]
        )
    plsc.subcore_barrier()


def _copy_out_acc(acc, out_hbm, c, s):
    plsc.subcore_barrier()
    pltpu.sync_copy(
        acc.at[pl.ds(s * ROWS_PER_TILE, ROWS_PER_TILE)],
        out_hbm.at[c, pl.ds(s * ROWS_PER_TILE, ROWS_PER_TILE)],
    )


def _make_seg_sum(chunks):
    @functools.partial(
        pl.kernel,
        out_type=jax.ShapeDtypeStruct((NC, NPAD, D), jnp.float32),
        mesh=_mesh(),
        scratch_types=[
            pltpu.VMEM((chunks, CHUNK), jnp.int32),
            pltpu.VMEM((chunks, CHUNK), jnp.int32),
            pltpu.VMEM((CHUNK, D), jnp.float32),
            pltpu.VMEM_SHARED((NPAD, D), jnp.float32),
            pltpu.SemaphoreType.DMA,
        ],
    )
    def seg_sum(table_hbm, src_hbm, dst_hbm, out_hbm, src_v, dst_v, rows_v, acc, sem):
        c = lax.axis_index("c")
        s = lax.axis_index("s")
        wid = c * NS + s
        pltpu.sync_copy(src_hbm.at[wid], src_v)
        pltpu.sync_copy(dst_hbm.at[wid], dst_v)
        _zero_acc_and_barrier(rows_v, acc, s)

        def body(j, carry):
            pltpu.async_copy(table_hbm.at[src_v.at[j]], rows_v, sem).wait()
            pltpu.sync_copy(rows_v, acc.at[dst_v.at[j]], add=True)
            return carry

        lax.fori_loop(0, chunks, body, 0)
        _copy_out_acc(acc, out_hbm, c, s)

    return seg_sum


def _make_deg(chunks):
    @functools.partial(
        pl.kernel,
        out_type=jax.ShapeDtypeStruct((NC, NPAD, D), jnp.float32),
        mesh=_mesh(),
        scratch_types=[
            pltpu.VMEM((chunks, CHUNK), jnp.int32),
            pltpu.VMEM((CHUNK, D), jnp.float32),
            pltpu.VMEM_SHARED((NPAD, D), jnp.float32),
        ],
    )
    def deg(dst_hbm, out_hbm, dst_v, rows_v, acc):
        c = lax.axis_index("c")
        s = lax.axis_index("s")
        wid = c * NS + s
        pltpu.sync_copy(dst_hbm.at[wid], dst_v)
        _zero_acc_and_barrier(rows_v, acc, s)
        _fill(rows_v, 1.0)

        def body(j, carry):
            pltpu.sync_copy(rows_v, acc.at[dst_v.at[j]], add=True)
            return carry

        lax.fori_loop(0, chunks, body, 0)
        _copy_out_acc(acc, out_hbm, c, s)

    return deg


def _row_spec():
    return pl.BlockSpec((BLK, D), lambda i: (i, 0))


def _full_spec():
    return pl.BlockSpec((D, D), lambda i: (0, 0))


def _vec_spec():
    return pl.BlockSpec((1, D), lambda i: (0, 0))


def _tc_first(xp, W1, d0, d1):
    def body(x_ref, w_ref, d0_ref, d1_ref, dis_ref, zp_ref):
        dis = lax.rsqrt(1.0 + d0_ref[...] + d1_ref[...])
        z = jnp.dot(x_ref[...], w_ref[...], preferred_element_type=jnp.float32)
        dis_ref[...] = dis
        zp_ref[...] = dis * z

    return pl.pallas_call(
        body,
        grid=(GRID,),
        in_specs=[_row_spec(), _full_spec(), _row_spec(), _row_spec()],
        out_specs=[_row_spec(), _row_spec()],
        out_shape=[jax.ShapeDtypeStruct((NPAD, D), jnp.float32)] * 2,
    )(xp, W1, d0, d1)


def _tc_mid(p0, p1, zp, dis_b, b, g, be, Wn):
    def body(p0_ref, p1_ref, zp_ref, dis_ref, b_ref, g_ref, be_ref, w_ref, out_ref):
        dis = dis_ref[...]
        u = dis * (p0_ref[...] + p1_ref[...] + zp_ref[...]) + b_ref[...]
        t = jnp.maximum(u * (g_ref[...] * BN_C) + be_ref[...], 0.0)
        out_ref[...] = dis * jnp.dot(
            t, w_ref[...], preferred_element_type=jnp.float32
        )

    return pl.pallas_call(
        body,
        grid=(GRID,),
        in_specs=[_row_spec(), _row_spec(), _row_spec(), _row_spec(),
                  _vec_spec(), _vec_spec(), _vec_spec(), _full_spec()],
        out_specs=_row_spec(),
        out_shape=jax.ShapeDtypeStruct((NPAD, D), jnp.float32),
    )(p0, p1, zp, dis_b, b.reshape(1, D), g.reshape(1, D), be.reshape(1, D), Wn)


def _tc_final(p0, p1, zp, dis_b, b, g, be, fcW, fcb):
    def body(p0_ref, p1_ref, zp_ref, dis_ref, b_ref, g_ref, be_ref, w_ref,
             fcb_ref, out_ref):
        dis = dis_ref[...]
        u = dis * (p0_ref[...] + p1_ref[...] + zp_ref[...]) + b_ref[...]
        t = jnp.maximum(u * (g_ref[...] * BN_C) + be_ref[...], 0.0)
        out_ref[...] = (
            jnp.dot(t, w_ref[...], preferred_element_type=jnp.float32)
            + fcb_ref[...]
        )

    return pl.pallas_call(
        body,
        grid=(GRID,),
        in_specs=[_row_spec(), _row_spec(), _row_spec(), _row_spec(),
                  _vec_spec(), _vec_spec(), _vec_spec(), _full_spec(), _vec_spec()],
        out_specs=_row_spec(),
        out_shape=jax.ShapeDtypeStruct((NPAD, D), jnp.float32),
    )(p0, p1, zp, dis_b, b.reshape(1, D), g.reshape(1, D), be.reshape(1, D),
      fcW, fcb.reshape(1, D))


def kernel(x, edge_index, W1, b1, g1, be1, W2, b2, g2, be2, W3, b3, g3, be3,
           fcW, fcb):
    E = edge_index.shape[1]
    chunks = -(-E // (NW * CHUNK))
    e_pad = chunks * NW * CHUNK
    pad = e_pad - E
    src = jnp.concatenate(
        [edge_index[0], jnp.zeros((pad,), jnp.int32)]).reshape(NW, chunks, CHUNK)
    dst = jnp.concatenate(
        [edge_index[1], jnp.full((pad,), N, jnp.int32)]).reshape(NW, chunks, CHUNK)
    xp = jnp.pad(x, ((0, NPAD - N), (0, 0)))

    seg_sum = _make_seg_sum(chunks)
    degP = _make_deg(chunks)(dst)
    dis_b, zp1 = _tc_first(xp, W1, degP[0], degP[1])
    a1 = seg_sum(zp1, src, dst)
    zp2 = _tc_mid(a1[0], a1[1], zp1, dis_b, b1, g1, be1, W2)
    a2 = seg_sum(zp2, src, dst)
    zp3 = _tc_mid(a2[0], a2[1], zp2, dis_b, b2, g2, be2, W3)
    a3 = seg_sum(zp3, src, dst)
    y = _tc_final(a3[0], a3[1], zp3, dis_b, b3, g3, be3, fcW, fcb)
    return y[:N]
